# Initial kernel scaffold; baseline (speedup 1.0000x reference)
import jax
import jax.numpy as jnp
from jax.experimental import pallas as pl


def kernel(x, edge_index, W1, b1, W2, b2, lin_W, lin_b, gamma, beta):
    raise NotImplementedError("write your pallas kernel here")


# speedup vs baseline: 36.2355x; 36.2355x over previous
import functools

import jax
import jax.numpy as jnp
from jax import lax
from jax.experimental import pallas as pl
from jax.experimental.pallas import tpu as pltpu
from jax.experimental.pallas import tpu_sc as plsc

_NC = 2
_NS = 16
_NW = _NC * _NS
_H = 16
_ALPHA = 0.1
_BNEPS = 1e-5
_KHOPS = 10


@functools.lru_cache(maxsize=None)
def _deg_kernel(np_pad: int, ib: int):
    rpt = np_pad // _NS

    def body(dst_rows, ones_rows, zrows, parts, acc, didx, ones_v):
        c = lax.axis_index("c")
        s = lax.axis_index("s")
        wid = s * _NC + c
        pltpu.sync_copy(zrows.at[pl.ds(s * rpt, rpt)],
                        acc.at[pl.ds(s * rpt, rpt)])
        pltpu.sync_copy(dst_rows.at[pl.ds(wid * ib, ib)], didx)
        pltpu.sync_copy(ones_rows, ones_v)
        plsc.subcore_barrier()

        def step(r, carry):
            pltpu.sync_copy(ones_v, acc.at[didx.at[r]], add=True)
            return carry

        lax.fori_loop(0, ib, step, None)
        plsc.subcore_barrier()
        pltpu.sync_copy(acc.at[pl.ds(s * rpt, rpt)],
                        parts.at[c, pl.ds(s * rpt, rpt)])

    return pl.kernel(
        body,
        out_type=jax.ShapeDtypeStruct((_NC, np_pad, _H), jnp.float32),
        mesh=plsc.VectorSubcoreMesh(core_axis_name="c", subcore_axis_name="s"),
        compiler_params=pltpu.CompilerParams(use_tc_tiling_on_sc=False),
        scratch_types=[
            pltpu.VMEM_SHARED((np_pad, _H), jnp.float32),
            pltpu.VMEM((ib, 128), jnp.int32),
            pltpu.VMEM((128, _H), jnp.float32),
        ],
    )


@functools.lru_cache(maxsize=None)
def _agg_kernel(np_pad: int, ib: int):
    rpt = np_pad // _NS

    def body(src_rows, dst_rows, table, zrows, parts, acc, sidx, didx, rows,
             gsem):
        c = lax.axis_index("c")
        s = lax.axis_index("s")
        wid = s * _NC + c
        pltpu.sync_copy(zrows.at[pl.ds(s * rpt, rpt)],
                        acc.at[pl.ds(s * rpt, rpt)])
        pltpu.sync_copy(src_rows.at[pl.ds(wid * ib, ib)], sidx)
        pltpu.sync_copy(dst_rows.at[pl.ds(wid * ib, ib)], didx)
        plsc.subcore_barrier()

        pltpu.async_copy(table.at[sidx.at[0]], rows.at[0], gsem.at[0])

        def step(r, carry):
            p = lax.rem(r, 2)
            pltpu.make_async_copy(table.at[sidx.at[r]], rows.at[p],
                                  gsem.at[p]).wait()

            @pl.when(r + 1 < ib)
            def _issue():
                pltpu.async_copy(table.at[sidx.at[r + 1]], rows.at[1 - p],
                                 gsem.at[1 - p])

            pltpu.sync_copy(rows.at[p], acc.at[didx.at[r]], add=True)
            return carry

        lax.fori_loop(0, ib, step, None)
        plsc.subcore_barrier()
        pltpu.sync_copy(acc.at[pl.ds(s * rpt, rpt)],
                        parts.at[c, pl.ds(s * rpt, rpt)])

    return pl.kernel(
        body,
        out_type=jax.ShapeDtypeStruct((_NC, np_pad, _H), jnp.float32),
        mesh=plsc.VectorSubcoreMesh(core_axis_name="c", subcore_axis_name="s"),
        compiler_params=pltpu.CompilerParams(use_tc_tiling_on_sc=False),
        scratch_types=[
            pltpu.VMEM_SHARED((np_pad, _H), jnp.float32),
            pltpu.VMEM((ib, 128), jnp.int32),
            pltpu.VMEM((ib, 128), jnp.int32),
            pltpu.VMEM((2, 128, _H), jnp.float32),
            pltpu.SemaphoreType.DMA((2,)),
        ],
    )


@functools.lru_cache(maxsize=None)
def _tc1(np_pad: int):
    def body(xp_ref, w1_ref, degp_ref, t1_ref, dinv_ref):
        u = jnp.dot(xp_ref[...], w1_ref[...],
                    preferred_element_type=jnp.float32)
        dp = degp_ref[...]
        deg = dp[0, :, 0:1] + dp[1, :, 0:1] + 1.0
        dinv = lax.rsqrt(deg)
        t1_ref[...] = u * dinv
        dinv_ref[...] = jnp.broadcast_to(dinv, (np_pad, _H))

    return pl.pallas_call(
        body,
        out_shape=(
            jax.ShapeDtypeStruct((np_pad, _H), jnp.float32),
            jax.ShapeDtypeStruct((np_pad, _H), jnp.float32),
        ),
    )


@functools.lru_cache(maxsize=None)
def _tc2(np_pad: int, n: int):
    pad = np_pad - n

    def body(p_ref, t1_ref, dinv_ref, b1_ref, gam_ref, bet_ref, lw_ref,
             lb_ref, t2_ref):
        p = p_ref[...]
        agg = (p[0] + p[1] + t1_ref[...]) * dinv_ref[...]
        h = jnp.maximum(agg[:n] + b1_ref[...], 0.0)
        mean = jnp.mean(h, axis=0, keepdims=True)
        var = jnp.mean((h - mean) ** 2, axis=0, keepdims=True)
        h = (h - mean) * lax.rsqrt(var + _BNEPS) * gam_ref[...] + bet_ref[...]
        lw = lw_ref[...]
        lb = lb_ref[...]
        for _ in range(_KHOPS):
            z = jnp.dot(h, lw, preferred_element_type=jnp.float32) + lb
            h = _ALPHA * jnp.maximum(z, 0.0) + (1.0 - _ALPHA) * h
        t2_ref[pl.ds(0, n), :] = h * dinv_ref[pl.ds(0, n), :]
        t2_ref[pl.ds(n, pad), :] = jnp.zeros((pad, _H), jnp.float32)

    return pl.pallas_call(
        body,
        out_shape=jax.ShapeDtypeStruct((np_pad, _H), jnp.float32),
    )


@functools.lru_cache(maxsize=None)
def _tc3(np_pad: int, n: int, c_dim: int):
    def body(q_ref, t2_ref, dinv_ref, w2_ref, b2_ref, out_ref):
        q = q_ref[...]
        v = (q[0] + q[1] + t2_ref[...]) * dinv_ref[...]
        logits = jnp.dot(v[:n], w2_ref[...],
                         preferred_element_type=jnp.float32) + b2_ref[...]
        m = jnp.max(logits, axis=1, keepdims=True)
        z = logits - m
        out_ref[...] = z - jnp.log(jnp.sum(jnp.exp(z), axis=1, keepdims=True))

    return pl.pallas_call(
        body,
        out_shape=jax.ShapeDtypeStruct((n, c_dim), jnp.float32),
    )


def kernel(x, edge_index, W1, b1, W2, b2, lin_W, lin_b, gamma, beta):
    n, d = x.shape
    e = edge_index.shape[1]
    c_dim = W2.shape[1]

    np_pad = ((n + 64 + 127) // 128) * 128
    pad_rows = np_pad - n
    epw = -(-e // (_NW * 1024)) * 1024
    ep = epw * _NW
    ib = epw // 128

    src = edge_index[0]
    dst = edge_index[1]
    pad_idx = (n + jnp.arange(ep - e, dtype=jnp.int32) % pad_rows).astype(
        jnp.int32)
    srcp = jnp.concatenate([src, pad_idx]).reshape(ep // 128, 128)
    dstp = jnp.concatenate([dst, pad_idx]).reshape(ep // 128, 128)

    zrows = jnp.zeros((np_pad, _H), jnp.float32)
    ones_rows = jnp.ones((128, _H), jnp.float32)
    xp = jnp.concatenate(
        [x, jnp.zeros((pad_rows, d), jnp.float32)], axis=0)

    degp = _deg_kernel(np_pad, ib)(dstp, ones_rows, zrows)
    t1, dinv = _tc1(np_pad)(xp, W1, degp)
    p1 = _agg_kernel(np_pad, ib)(srcp, dstp, t1, zrows)
    t2 = _tc2(np_pad, n)(p1, t1, dinv, b1.reshape(1, _H),
                         gamma.reshape(1, _H), beta.reshape(1, _H), lin_W,
                         lin_b.reshape(1, _H))
    q = _agg_kernel(np_pad, ib)(srcp, dstp, t2, zrows)
    out = _tc3(np_pad, n, c_dim)(q, t2, dinv, W2, b2.reshape(1, c_dim))
    return out


# speedup vs baseline: 57.2942x; 1.5812x over previous
import functools

import jax
import jax.numpy as jnp
from jax import lax
from jax.experimental import pallas as pl
from jax.experimental.pallas import tpu as pltpu
from jax.experimental.pallas import tpu_sc as plsc

_NC = 2
_NS = 16
_NW = _NC * _NS
_H = 16
_ALPHA = 0.1
_BNEPS = 1e-5
_KHOPS = 10


@functools.lru_cache(maxsize=None)
def _deg_kernel(np_pad: int, ib: int):
    rpt = np_pad // _NS

    nb = 4

    def body(dst_rows, ones_rows, zrows, parts, acc, didx, ones_v, ssem):
        c = lax.axis_index("c")
        s = lax.axis_index("s")
        wid = s * _NC + c
        pltpu.sync_copy(zrows.at[pl.ds(s * rpt, rpt)],
                        acc.at[pl.ds(s * rpt, rpt)])
        pltpu.sync_copy(dst_rows.at[pl.ds(wid * ib, ib)], didx)
        pltpu.sync_copy(ones_rows, ones_v)
        plsc.subcore_barrier()

        def step(r, carry):
            p = lax.rem(r, nb)

            @pl.when(r >= nb)
            def _drain():
                pltpu.make_async_copy(ones_v, acc.at[didx.at[r - nb]],
                                      ssem.at[p]).wait()

            pltpu.async_copy(ones_v, acc.at[didx.at[r]], ssem.at[p], add=True)
            return carry

        lax.fori_loop(0, ib, step, None)
        for j in range(nb):
            pltpu.make_async_copy(ones_v, acc.at[didx.at[ib - nb + j]],
                                  ssem.at[j]).wait()
        plsc.subcore_barrier()
        pltpu.sync_copy(acc.at[pl.ds(s * rpt, rpt)],
                        parts.at[c, pl.ds(s * rpt, rpt)])

    return pl.kernel(
        body,
        out_type=jax.ShapeDtypeStruct((_NC, np_pad, _H), jnp.float32),
        mesh=plsc.VectorSubcoreMesh(core_axis_name="c", subcore_axis_name="s"),
        compiler_params=pltpu.CompilerParams(use_tc_tiling_on_sc=False),
        scratch_types=[
            pltpu.VMEM_SHARED((np_pad, _H), jnp.float32),
            pltpu.VMEM((ib, 128), jnp.int32),
            pltpu.VMEM((128, _H), jnp.float32),
            pltpu.SemaphoreType.DMA((nb,)),
        ],
    )


@functools.lru_cache(maxsize=None)
def _agg_kernel(np_pad: int, ib: int):
    rpt = np_pad // _NS

    nbuf = 8
    look = 4

    def body(src_rows, dst_rows, table, zrows, parts, tbl_sh, acc, sidx, didx,
             rows, gsem, ssem):
        c = lax.axis_index("c")
        s = lax.axis_index("s")
        wid = s * _NC + c
        pltpu.sync_copy(zrows.at[pl.ds(s * rpt, rpt)],
                        acc.at[pl.ds(s * rpt, rpt)])
        pltpu.sync_copy(table.at[pl.ds(s * rpt, rpt)],
                        tbl_sh.at[pl.ds(s * rpt, rpt)])
        pltpu.sync_copy(src_rows.at[pl.ds(wid * ib, ib)], sidx)
        pltpu.sync_copy(dst_rows.at[pl.ds(wid * ib, ib)], didx)
        plsc.subcore_barrier()

        for j in range(look):
            pltpu.async_copy(tbl_sh.at[sidx.at[j]], rows.at[j], gsem.at[j])

        def step(r, carry):
            p = lax.rem(r, nbuf)
            pltpu.make_async_copy(tbl_sh.at[sidx.at[r]], rows.at[p],
                                  gsem.at[p]).wait()
            pltpu.async_copy(rows.at[p], acc.at[didx.at[r]], ssem.at[p],
                             add=True)
            g = r + look

            @pl.when(g < ib)
            def _refill():
                q = lax.rem(g, nbuf)

                @pl.when(r >= look)
                def _drain():
                    pltpu.make_async_copy(rows.at[q],
                                          acc.at[didx.at[g - nbuf]],
                                          ssem.at[q]).wait()

                pltpu.async_copy(tbl_sh.at[sidx.at[g]], rows.at[q],
                                 gsem.at[q])

            return carry

        lax.fori_loop(0, ib, step, None)
        for j in range(nbuf):
            r_last = ib - nbuf + j
            pltpu.make_async_copy(rows.at[j], acc.at[didx.at[r_last]],
                                  ssem.at[j]).wait()
        plsc.subcore_barrier()
        pltpu.sync_copy(acc.at[pl.ds(s * rpt, rpt)],
                        parts.at[c, pl.ds(s * rpt, rpt)])

    return pl.kernel(
        body,
        out_type=jax.ShapeDtypeStruct((_NC, np_pad, _H), jnp.float32),
        mesh=plsc.VectorSubcoreMesh(core_axis_name="c", subcore_axis_name="s"),
        compiler_params=pltpu.CompilerParams(use_tc_tiling_on_sc=False),
        scratch_types=[
            pltpu.VMEM_SHARED((np_pad, _H), jnp.float32),
            pltpu.VMEM_SHARED((np_pad, _H), jnp.float32),
            pltpu.VMEM((ib, 128), jnp.int32),
            pltpu.VMEM((ib, 128), jnp.int32),
            pltpu.VMEM((nbuf, 128, _H), jnp.float32),
            pltpu.SemaphoreType.DMA((nbuf,)),
            pltpu.SemaphoreType.DMA((nbuf,)),
        ],
    )


@functools.lru_cache(maxsize=None)
def _tc1(np_pad: int):
    def body(xp_ref, w1_ref, degp_ref, t1_ref, dinv_ref):
        u = jnp.dot(xp_ref[...], w1_ref[...],
                    preferred_element_type=jnp.float32)
        dp = degp_ref[...]
        deg = dp[0, :, 0:1] + dp[1, :, 0:1] + 1.0
        dinv = lax.rsqrt(deg)
        t1_ref[...] = u * dinv
        dinv_ref[...] = jnp.broadcast_to(dinv, (np_pad, _H))

    return pl.pallas_call(
        body,
        out_shape=(
            jax.ShapeDtypeStruct((np_pad, _H), jnp.float32),
            jax.ShapeDtypeStruct((np_pad, _H), jnp.float32),
        ),
    )


@functools.lru_cache(maxsize=None)
def _tc2(np_pad: int, n: int):
    pad = np_pad - n

    def body(p_ref, t1_ref, dinv_ref, b1_ref, gam_ref, bet_ref, lw_ref,
             lb_ref, t2_ref):
        p = p_ref[...]
        agg = (p[0] + p[1] + t1_ref[...]) * dinv_ref[...]
        h = jnp.maximum(agg[:n] + b1_ref[...], 0.0)
        mean = jnp.mean(h, axis=0, keepdims=True)
        var = jnp.mean((h - mean) ** 2, axis=0, keepdims=True)
        h = (h - mean) * lax.rsqrt(var + _BNEPS) * gam_ref[...] + bet_ref[...]
        lw = lw_ref[...]
        lb = lb_ref[...]
        for _ in range(_KHOPS):
            z = jnp.dot(h, lw, preferred_element_type=jnp.float32) + lb
            h = _ALPHA * jnp.maximum(z, 0.0) + (1.0 - _ALPHA) * h
        t2_ref[pl.ds(0, n), :] = h * dinv_ref[pl.ds(0, n), :]
        t2_ref[pl.ds(n, pad), :] = jnp.zeros((pad, _H), jnp.float32)

    return pl.pallas_call(
        body,
        out_shape=jax.ShapeDtypeStruct((np_pad, _H), jnp.float32),
    )


@functools.lru_cache(maxsize=None)
def _tc3(np_pad: int, n: int, c_dim: int):
    def body(q_ref, t2_ref, dinv_ref, w2_ref, b2_ref, out_ref):
        q = q_ref[...]
        v = (q[0] + q[1] + t2_ref[...]) * dinv_ref[...]
        logits = jnp.dot(v[:n], w2_ref[...],
                         preferred_element_type=jnp.float32) + b2_ref[...]
        m = jnp.max(logits, axis=1, keepdims=True)
        z = logits - m
        out_ref[...] = z - jnp.log(jnp.sum(jnp.exp(z), axis=1, keepdims=True))

    return pl.pallas_call(
        body,
        out_shape=jax.ShapeDtypeStruct((n, c_dim), jnp.float32),
    )


def kernel(x, edge_index, W1, b1, W2, b2, lin_W, lin_b, gamma, beta):
    n, d = x.shape
    e = edge_index.shape[1]
    c_dim = W2.shape[1]

    np_pad = ((n + 64 + 127) // 128) * 128
    pad_rows = np_pad - n
    epw = -(-e // (_NW * 1024)) * 1024
    ep = epw * _NW
    ib = epw // 128

    src = edge_index[0]
    dst = edge_index[1]
    pad_idx = (n + jnp.arange(ep - e, dtype=jnp.int32) % pad_rows).astype(
        jnp.int32)
    srcp = jnp.concatenate([src, pad_idx]).reshape(ep // 128, 128)
    dstp = jnp.concatenate([dst, pad_idx]).reshape(ep // 128, 128)

    zrows = jnp.zeros((np_pad, _H), jnp.float32)
    ones_rows = jnp.ones((128, _H), jnp.float32)
    xp = jnp.concatenate(
        [x, jnp.zeros((pad_rows, d), jnp.float32)], axis=0)

    degp = _deg_kernel(np_pad, ib)(dstp, ones_rows, zrows)
    t1, dinv = _tc1(np_pad)(xp, W1, degp)
    p1 = _agg_kernel(np_pad, ib)(srcp, dstp, t1, zrows)
    t2 = _tc2(np_pad, n)(p1, t1, dinv, b1.reshape(1, _H),
                         gamma.reshape(1, _H), beta.reshape(1, _H), lin_W,
                         lin_b.reshape(1, _H))
    q = _agg_kernel(np_pad, ib)(srcp, dstp, t2, zrows)
    out = _tc3(np_pad, n, c_dim)(q, t2, dinv, W2, b2.reshape(1, c_dim))
    return out


# speedup vs baseline: 78.0259x; 1.3618x over previous
import functools

import jax
import jax.numpy as jnp
from jax import lax
from jax.experimental import pallas as pl
from jax.experimental.pallas import tpu as pltpu
from jax.experimental.pallas import tpu_sc as plsc

_NC = 2
_NS = 16
_NW = _NC * _NS
_H = 16
_ALPHA = 0.1
_BNEPS = 1e-5
_KHOPS = 10


@functools.lru_cache(maxsize=None)
def _deg_kernel(np_pad: int, ib: int):
    rpt = np_pad // _NS
    nb = 4

    def body(dst_rows, ones_rows, zrows, parts, acc, didx, ones_v, ssem):
        c = lax.axis_index("c")
        s = lax.axis_index("s")
        wid = s * _NC + c
        pltpu.sync_copy(zrows.at[pl.ds(s * rpt, rpt)],
                        acc.at[pl.ds(s * rpt, rpt)])
        pltpu.sync_copy(dst_rows.at[pl.ds(wid * ib, ib)], didx)
        pltpu.sync_copy(ones_rows, ones_v)
        plsc.subcore_barrier()

        def step(r, carry):
            p = lax.rem(r, nb)

            @pl.when(r >= nb)
            def _drain():
                pltpu.make_async_copy(ones_v, acc.at[didx.at[r - nb]],
                                      ssem.at[p]).wait()

            pltpu.async_copy(ones_v, acc.at[didx.at[r]], ssem.at[p], add=True)
            return carry

        lax.fori_loop(0, ib, step, None)
        for j in range(nb):
            pltpu.make_async_copy(ones_v, acc.at[didx.at[ib - nb + j]],
                                  ssem.at[j]).wait()
        plsc.subcore_barrier()
        pltpu.sync_copy(acc.at[pl.ds(s * rpt, rpt)],
                        parts.at[c, pl.ds(s * rpt, rpt)])

    return pl.kernel(
        body,
        out_type=jax.ShapeDtypeStruct((_NC, np_pad, _H), jnp.float32),
        mesh=plsc.VectorSubcoreMesh(core_axis_name="c", subcore_axis_name="s"),
        compiler_params=pltpu.CompilerParams(use_tc_tiling_on_sc=False),
        scratch_types=[
            pltpu.VMEM_SHARED((np_pad, _H), jnp.float32),
            pltpu.VMEM((ib, 128), jnp.int32),
            pltpu.VMEM((128, _H), jnp.float32),
            pltpu.SemaphoreType.DMA((nb,)),
        ],
    )


@functools.lru_cache(maxsize=None)
def _agg_kernel(np_pad: int, ib: int):
    rpt = np_pad // _NS
    nbuf = 8
    look = 4

    def body(src_rows, dst_rows, table, zrows, parts, tbl_sh, acc, sidx, didx,
             rows, gsem, ssem):
        c = lax.axis_index("c")
        s = lax.axis_index("s")
        wid = s * _NC + c
        pltpu.sync_copy(zrows.at[pl.ds(s * rpt, rpt)],
                        acc.at[pl.ds(s * rpt, rpt)])
        pltpu.sync_copy(table.at[pl.ds(s * rpt, rpt)],
                        tbl_sh.at[pl.ds(s * rpt, rpt)])
        pltpu.sync_copy(src_rows.at[pl.ds(wid * ib, ib)], sidx)
        pltpu.sync_copy(dst_rows.at[pl.ds(wid * ib, ib)], didx)
        plsc.subcore_barrier()

        for j in range(look):
            pltpu.async_copy(tbl_sh.at[sidx.at[j]], rows.at[j], gsem.at[j])

        def step(r, carry):
            p = lax.rem(r, nbuf)
            pltpu.make_async_copy(tbl_sh.at[sidx.at[r]], rows.at[p],
                                  gsem.at[p]).wait()
            pltpu.async_copy(rows.at[p], acc.at[didx.at[r]], ssem.at[p],
                             add=True)
            g = r + look

            @pl.when(g < ib)
            def _refill():
                q = lax.rem(g, nbuf)

                @pl.when(r >= look)
                def _drain():
                    pltpu.make_async_copy(rows.at[q],
                                          acc.at[didx.at[g - nbuf]],
                                          ssem.at[q]).wait()

                pltpu.async_copy(tbl_sh.at[sidx.at[g]], rows.at[q],
                                 gsem.at[q])

            return carry

        lax.fori_loop(0, ib, step, None)
        for j in range(nbuf):
            r_last = ib - nbuf + j
            pltpu.make_async_copy(rows.at[j], acc.at[didx.at[r_last]],
                                  ssem.at[j]).wait()
        plsc.subcore_barrier()
        pltpu.sync_copy(acc.at[pl.ds(s * rpt, rpt)],
                        parts.at[c, pl.ds(s * rpt, rpt)])

    return pl.kernel(
        body,
        out_type=jax.ShapeDtypeStruct((_NC, np_pad, _H), jnp.float32),
        mesh=plsc.VectorSubcoreMesh(core_axis_name="c", subcore_axis_name="s"),
        compiler_params=pltpu.CompilerParams(use_tc_tiling_on_sc=False),
        scratch_types=[
            pltpu.VMEM_SHARED((np_pad, _H), jnp.float32),
            pltpu.VMEM_SHARED((np_pad, _H), jnp.float32),
            pltpu.VMEM((ib, 128), jnp.int32),
            pltpu.VMEM((ib, 128), jnp.int32),
            pltpu.VMEM((nbuf, 128, _H), jnp.float32),
            pltpu.SemaphoreType.DMA((nbuf,)),
            pltpu.SemaphoreType.DMA((nbuf,)),
        ],
    )


def _group_first_bcast(v):
    i = lax.broadcasted_iota(jnp.int32, (128, 128), 0)
    j = lax.broadcasted_iota(jnp.int32, (128, 128), 1)
    sel = ((j // _H) * _H == i).astype(jnp.float32)
    return jnp.dot(v, sel, preferred_element_type=jnp.float32)


def _group_sum16(row128):
    i = lax.broadcasted_iota(jnp.int32, (128, _H), 0)
    j = lax.broadcasted_iota(jnp.int32, (128, _H), 1)
    g = (i % _H == j).astype(jnp.float32)
    return jnp.dot(row128, g, preferred_element_type=jnp.float32)


def _bcast128(row16):
    i = lax.broadcasted_iota(jnp.int32, (_H, 128), 0)
    j = lax.broadcasted_iota(jnp.int32, (_H, 128), 1)
    g = (j % _H == i).astype(jnp.float32)
    return jnp.dot(row16, g, preferred_element_type=jnp.float32)


def _node_mask(nrb, n):
    r = lax.broadcasted_iota(jnp.int32, (nrb, 128), 0)
    l = lax.broadcasted_iota(jnp.int32, (nrb, 128), 1)
    node = (l // _H) * nrb + r
    return (node < n).astype(jnp.float32)


@functools.lru_cache(maxsize=None)
def _tc1(np_pad: int, n: int):
    nrb = np_pad // 8

    def body(x_ref, w1_ref, degp_ref, t1_ref, dinv_ref):
        u = jnp.dot(x_ref[...], w1_ref[...],
                    preferred_element_type=jnp.float32)
        u = jnp.concatenate(
            [u, jnp.zeros((np_pad - n, _H), jnp.float32)], axis=0)
        u_pk = jnp.concatenate(
            [u[a * nrb:(a + 1) * nrb] for a in range(8)], axis=1)
        dsum = degp_ref[0] + degp_ref[1]
        dinv = lax.rsqrt(_group_first_bcast(dsum) + 1.0)
        t1_ref[...] = u_pk * dinv
        dinv_ref[...] = dinv

    return pl.pallas_call(
        body,
        out_shape=(
            jax.ShapeDtypeStruct((nrb, 128), jnp.float32),
            jax.ShapeDtypeStruct((nrb, 128), jnp.float32),
        ),
    )


@functools.lru_cache(maxsize=None)
def _tc2(np_pad: int, n: int):
    nrb = np_pad // 8

    def body(p_ref, t1_ref, dinv_ref, b1t_ref, gamt_ref, bett_ref, lwk_ref,
             lbt_ref, t2_ref):
        mask = _node_mask(nrb, n)
        dinv = dinv_ref[...]
        agg = (p_ref[0] + p_ref[1] + t1_ref[...]) * dinv
        h = jnp.maximum(agg + b1t_ref[...], 0.0) * mask
        s = jnp.sum(h, axis=0, keepdims=True)
        mean = _bcast128(_group_sum16(s) * (1.0 / n))
        hc = (h - mean) * mask
        s2 = jnp.sum(hc * hc, axis=0, keepdims=True)
        var = _bcast128(_group_sum16(s2) * (1.0 / n))
        h = hc * lax.rsqrt(var + _BNEPS) * gamt_ref[...] + bett_ref[...]
        lwk = lwk_ref[...]
        lbt = lbt_ref[...]
        for _ in range(_KHOPS):
            z = jnp.dot(h, lwk, preferred_element_type=jnp.float32) + lbt
            h = _ALPHA * jnp.maximum(z, 0.0) + (1.0 - _ALPHA) * h
        t2_ref[...] = h * dinv * mask

    return pl.pallas_call(
        body,
        out_shape=jax.ShapeDtypeStruct((nrb, 128), jnp.float32),
    )


@functools.lru_cache(maxsize=None)
def _tc3(np_pad: int, n: int, c_dim: int):
    nrb = np_pad // 8

    def body(q_ref, t2_ref, dinv_ref, w2_ref, b2_ref, out_ref):
        v = (q_ref[0] + q_ref[1] + t2_ref[...]) * dinv_ref[...]
        w2 = w2_ref[...]
        b2 = b2_ref[...]
        for a in range(8):
            lo = a * nrb
            if lo >= n:
                break
            rows = min(nrb, n - lo)
            va = v[:, a * _H:(a + 1) * _H]
            logits = jnp.dot(va, w2,
                             preferred_element_type=jnp.float32) + b2
            m = jnp.max(logits, axis=1, keepdims=True)
            z = logits - m
            ls = z - jnp.log(jnp.sum(jnp.exp(z), axis=1, keepdims=True))
            out_ref[pl.ds(lo, rows), :] = ls[:rows]

    return pl.pallas_call(
        body,
        out_shape=jax.ShapeDtypeStruct((n, c_dim), jnp.float32),
    )


def kernel(x, edge_index, W1, b1, W2, b2, lin_W, lin_b, gamma, beta):
    n, d = x.shape
    e = edge_index.shape[1]
    c_dim = W2.shape[1]

    np_pad = ((n + 64 + 127) // 128) * 128
    pad_rows = np_pad - n
    nrb = np_pad // 8
    epw = -(-e // (_NW * 1024)) * 1024
    ep = epw * _NW
    ib = epw // 128

    src = edge_index[0]
    dst = edge_index[1]
    pad_idx = (n + jnp.arange(ep - e, dtype=jnp.int32) % pad_rows).astype(
        jnp.int32)
    src = jnp.concatenate([src, pad_idx])
    dst = jnp.concatenate([dst, pad_idx])
    srcp = (8 * (src % nrb) + src // nrb).reshape(ep // 128, 128)
    dstp = (8 * (dst % nrb) + dst // nrb).reshape(ep // 128, 128)

    zrows = jnp.zeros((np_pad, _H), jnp.float32)
    ones_rows = jnp.ones((128, _H), jnp.float32)

    kron8 = jnp.kron(jnp.eye(8, dtype=jnp.float32), lin_W)
    b1t = jnp.tile(b1, 8).reshape(1, 128)
    gamt = jnp.tile(gamma, 8).reshape(1, 128)
    bett = jnp.tile(beta, 8).reshape(1, 128)
    lbt = jnp.tile(lin_b, 8).reshape(1, 128)

    degp = _deg_kernel(np_pad, ib)(dstp, ones_rows, zrows)
    t1_pk, dinv_pk = _tc1(np_pad, n)(x, W1, degp.reshape(_NC, nrb, 128))
    p1 = _agg_kernel(np_pad, ib)(srcp, dstp, t1_pk.reshape(np_pad, _H),
                                 zrows)
    t2_pk = _tc2(np_pad, n)(p1.reshape(_NC, nrb, 128), t1_pk, dinv_pk, b1t,
                            gamt, bett, kron8, lbt)
    q = _agg_kernel(np_pad, ib)(srcp, dstp, t2_pk.reshape(np_pad, _H), zrows)
    out = _tc3(np_pad, n, c_dim)(q.reshape(_NC, nrb, 128), t2_pk, dinv_pk,
                                 W2, b2.reshape(1, c_dim))
    return out


# speedup vs baseline: 84.4769x; 1.0827x over previous
import functools

import jax
import jax.numpy as jnp
from jax import lax
from jax.experimental import pallas as pl
from jax.experimental.pallas import tpu as pltpu
from jax.experimental.pallas import tpu_sc as plsc

_NC = 2
_NS = 16
_NW = _NC * _NS
_H = 16
_ALPHA = 0.1
_BNEPS = 1e-5
_KHOPS = 10


@functools.lru_cache(maxsize=None)
def _deg_kernel(np_pad: int, ib: int):
    rpt = np_pad // _NS
    nb = 4

    def body(dst_rows, ones_rows, zrows, parts, acc, didx, ones_v, ssem):
        c = lax.axis_index("c")
        s = lax.axis_index("s")
        wid = s * _NC + c
        pltpu.sync_copy(zrows.at[pl.ds(s * rpt, rpt)],
                        acc.at[pl.ds(s * rpt, rpt)])
        pltpu.sync_copy(dst_rows.at[pl.ds(wid * ib, ib)], didx)
        pltpu.sync_copy(ones_rows, ones_v)
        plsc.subcore_barrier()

        def step(r, carry):
            p = lax.rem(r, nb)

            @pl.when(r >= nb)
            def _drain():
                pltpu.make_async_copy(ones_v, acc.at[didx.at[r - nb]],
                                      ssem.at[p]).wait()

            pltpu.async_copy(ones_v, acc.at[didx.at[r]], ssem.at[p], add=True)
            return carry

        lax.fori_loop(0, ib, step, None)
        for j in range(nb):
            pltpu.make_async_copy(ones_v, acc.at[didx.at[ib - nb + j]],
                                  ssem.at[j]).wait()
        plsc.subcore_barrier()
        pltpu.sync_copy(acc.at[pl.ds(s * rpt, rpt)],
                        parts.at[c, pl.ds(s * rpt, rpt)])

    return pl.kernel(
        body,
        out_type=jax.ShapeDtypeStruct((_NC, np_pad, _H), jnp.float32),
        mesh=plsc.VectorSubcoreMesh(core_axis_name="c", subcore_axis_name="s"),
        compiler_params=pltpu.CompilerParams(use_tc_tiling_on_sc=False),
        scratch_types=[
            pltpu.VMEM_SHARED((np_pad, _H), jnp.float32),
            pltpu.VMEM((ib, 128), jnp.int32),
            pltpu.VMEM((128, _H), jnp.float32),
            pltpu.SemaphoreType.DMA((nb,)),
        ],
    )


@functools.lru_cache(maxsize=None)
def _agg_kernel(np_pad: int, ib: int):
    rpt = np_pad // _NS
    nbuf = 8
    look = 4

    def body(src_rows, dst_rows, table, zrows, parts, tbl_sh, acc, sidx, didx,
             rows, gsem, ssem):
        c = lax.axis_index("c")
        s = lax.axis_index("s")
        wid = s * _NC + c
        pltpu.sync_copy(zrows.at[pl.ds(s * rpt, rpt)],
                        acc.at[pl.ds(s * rpt, rpt)])
        pltpu.sync_copy(table.at[pl.ds(s * rpt, rpt)],
                        tbl_sh.at[pl.ds(s * rpt, rpt)])
        pltpu.sync_copy(src_rows.at[pl.ds(wid * ib, ib)], sidx)
        pltpu.sync_copy(dst_rows.at[pl.ds(wid * ib, ib)], didx)
        plsc.subcore_barrier()

        for j in range(look):
            pltpu.async_copy(tbl_sh.at[sidx.at[j]], rows.at[j], gsem.at[j])

        def step(r, carry):
            p = lax.rem(r, nbuf)
            pltpu.make_async_copy(tbl_sh.at[sidx.at[r]], rows.at[p],
                                  gsem.at[p]).wait()
            pltpu.async_copy(rows.at[p], acc.at[didx.at[r]], ssem.at[p],
                             add=True)
            g = r + look

            @pl.when(g < ib)
            def _refill():
                q = lax.rem(g, nbuf)

                @pl.when(r >= look)
                def _drain():
                    pltpu.make_async_copy(rows.at[q],
                                          acc.at[didx.at[g - nbuf]],
                                          ssem.at[q]).wait()

                pltpu.async_copy(tbl_sh.at[sidx.at[g]], rows.at[q],
                                 gsem.at[q])

            return carry

        lax.fori_loop(0, ib, step, None)
        for j in range(nbuf):
            r_last = ib - nbuf + j
            pltpu.make_async_copy(rows.at[j], acc.at[didx.at[r_last]],
                                  ssem.at[j]).wait()
        plsc.subcore_barrier()
        pltpu.sync_copy(acc.at[pl.ds(s * rpt, rpt)],
                        parts.at[c, pl.ds(s * rpt, rpt)])

    return pl.kernel(
        body,
        out_type=jax.ShapeDtypeStruct((_NC, np_pad, _H), jnp.float32),
        mesh=plsc.VectorSubcoreMesh(core_axis_name="c", subcore_axis_name="s"),
        compiler_params=pltpu.CompilerParams(use_tc_tiling_on_sc=False),
        scratch_types=[
            pltpu.VMEM_SHARED((np_pad, _H), jnp.float32),
            pltpu.VMEM_SHARED((np_pad, _H), jnp.float32),
            pltpu.VMEM((ib, 128), jnp.int32),
            pltpu.VMEM((ib, 128), jnp.int32),
            pltpu.VMEM((nbuf, 128, _H), jnp.float32),
            pltpu.SemaphoreType.DMA((nbuf,)),
            pltpu.SemaphoreType.DMA((nbuf,)),
        ],
    )


def _group_first_bcast(v):
    i = lax.broadcasted_iota(jnp.int32, (128, 128), 0)
    j = lax.broadcasted_iota(jnp.int32, (128, 128), 1)
    sel = ((j // _H) * _H == i).astype(jnp.float32)
    return jnp.dot(v, sel, preferred_element_type=jnp.float32)


def _group_sum16(row128):
    i = lax.broadcasted_iota(jnp.int32, (128, _H), 0)
    j = lax.broadcasted_iota(jnp.int32, (128, _H), 1)
    g = (i % _H == j).astype(jnp.float32)
    return jnp.dot(row128, g, preferred_element_type=jnp.float32)


def _bcast128(row16):
    i = lax.broadcasted_iota(jnp.int32, (_H, 128), 0)
    j = lax.broadcasted_iota(jnp.int32, (_H, 128), 1)
    g = (j % _H == i).astype(jnp.float32)
    return jnp.dot(row16, g, preferred_element_type=jnp.float32)


def _node_mask(nrb, n):
    r = lax.broadcasted_iota(jnp.int32, (nrb, 128), 0)
    l = lax.broadcasted_iota(jnp.int32, (nrb, 128), 1)
    node = (l // _H) * nrb + r
    return (node < n).astype(jnp.float32)


@functools.lru_cache(maxsize=None)
def _prep_kernel(e: int, ep: int, n: int, nrb: int, blk: int):
    rows_per_blk = blk // 128
    grid = ep // blk

    def body(ei_ref, srcp_ref, dstp_ref):
        i = pl.program_id(0)
        gidx = i * blk + lax.broadcasted_iota(jnp.int32, (2, blk), 1)
        raw = ei_ref[...]
        v = jnp.where(gidx < e, raw, n + (gidx & 63))
        vf = v.astype(jnp.float32)
        q = jnp.floor((vf + 0.5) * (1.0 / nrb)).astype(jnp.int32)
        idx = 8 * (v - q * nrb) + q
        srcp_ref[...] = idx[0].reshape(rows_per_blk, 128)
        dstp_ref[...] = idx[1].reshape(rows_per_blk, 128)

    return pl.pallas_call(
        body,
        grid=(grid,),
        in_specs=[pl.BlockSpec((2, blk), lambda i: (0, i))],
        out_specs=(
            pl.BlockSpec((rows_per_blk, 128), lambda i: (i, 0)),
            pl.BlockSpec((rows_per_blk, 128), lambda i: (i, 0)),
        ),
        out_shape=(
            jax.ShapeDtypeStruct((ep // 128, 128), jnp.int32),
            jax.ShapeDtypeStruct((ep // 128, 128), jnp.int32),
        ),
    )


@functools.lru_cache(maxsize=None)
def _tc0(np_pad: int, n: int):
    nrb = np_pad // 8

    def body(x_ref, w1_ref, u_ref):
        u = jnp.dot(x_ref[...], w1_ref[...],
                    preferred_element_type=jnp.float32)
        u = jnp.concatenate(
            [u, jnp.zeros((np_pad - n, _H), jnp.float32)], axis=0)
        u_ref[...] = jnp.concatenate(
            [u[a * nrb:(a + 1) * nrb] for a in range(8)], axis=1)

    return pl.pallas_call(
        body,
        out_shape=jax.ShapeDtypeStruct((nrb, 128), jnp.float32),
    )


@functools.lru_cache(maxsize=None)
def _tc1(np_pad: int):
    nrb = np_pad // 8

    def body(u_ref, degp_ref, t1_ref, dinv_ref):
        dsum = degp_ref[0] + degp_ref[1]
        dinv = lax.rsqrt(_group_first_bcast(dsum) + 1.0)
        t1_ref[...] = u_ref[...] * dinv
        dinv_ref[...] = dinv

    return pl.pallas_call(
        body,
        out_shape=(
            jax.ShapeDtypeStruct((nrb, 128), jnp.float32),
            jax.ShapeDtypeStruct((nrb, 128), jnp.float32),
        ),
    )


@functools.lru_cache(maxsize=None)
def _tc2(np_pad: int, n: int):
    nrb = np_pad // 8

    def body(p_ref, t1_ref, dinv_ref, b1t_ref, gamt_ref, bett_ref, lwk_ref,
             lbt_ref, t2_ref):
        mask = _node_mask(nrb, n)
        dinv = dinv_ref[...]
        agg = (p_ref[0] + p_ref[1] + t1_ref[...]) * dinv
        h = jnp.maximum(agg + b1t_ref[...], 0.0) * mask
        s = jnp.sum(h, axis=0, keepdims=True)
        mean = _bcast128(_group_sum16(s) * (1.0 / n))
        hc = (h - mean) * mask
        s2 = jnp.sum(hc * hc, axis=0, keepdims=True)
        var = _bcast128(_group_sum16(s2) * (1.0 / n))
        h = hc * lax.rsqrt(var + _BNEPS) * gamt_ref[...] + bett_ref[...]
        lwk = lwk_ref[...]
        lbt = lbt_ref[...]
        for _ in range(_KHOPS):
            z = jnp.dot(h, lwk, preferred_element_type=jnp.float32) + lbt
            h = _ALPHA * jnp.maximum(z, 0.0) + (1.0 - _ALPHA) * h
        t2_ref[...] = h * dinv * mask

    return pl.pallas_call(
        body,
        out_shape=jax.ShapeDtypeStruct((nrb, 128), jnp.float32),
    )


@functools.lru_cache(maxsize=None)
def _tc3(np_pad: int, n: int, c_dim: int):
    nrb = np_pad // 8

    def body(q_ref, t2_ref, dinv_ref, w2_ref, b2_ref, out_ref):
        v = (q_ref[0] + q_ref[1] + t2_ref[...]) * dinv_ref[...]
        w2 = w2_ref[...]
        b2 = b2_ref[...]
        for a in range(8):
            lo = a * nrb
            if lo >= n:
                break
            rows = min(nrb, n - lo)
            va = v[:, a * _H:(a + 1) * _H]
            logits = jnp.dot(va, w2,
                             preferred_element_type=jnp.float32) + b2
            m = jnp.max(logits, axis=1, keepdims=True)
            z = logits - m
            ls = z - jnp.log(jnp.sum(jnp.exp(z), axis=1, keepdims=True))
            out_ref[pl.ds(lo, rows), :] = ls[:rows]

    return pl.pallas_call(
        body,
        out_shape=jax.ShapeDtypeStruct((n, c_dim), jnp.float32),
    )


def kernel(x, edge_index, W1, b1, W2, b2, lin_W, lin_b, gamma, beta):
    n, d = x.shape
    e = edge_index.shape[1]
    c_dim = W2.shape[1]

    np_pad = ((n + 64 + 127) // 128) * 128
    pad_rows = np_pad - n
    nrb = np_pad // 8
    epw = -(-e // (_NW * 1024)) * 1024
    ep = epw * _NW
    ib = epw // 128

    srcp, dstp = _prep_kernel(e, ep, n, nrb, ep // 10)(edge_index)

    zrows = jnp.zeros((np_pad, _H), jnp.float32)
    ones_rows = jnp.ones((128, _H), jnp.float32)

    kron8 = jnp.kron(jnp.eye(8, dtype=jnp.float32), lin_W)
    b1t = jnp.tile(b1, 8).reshape(1, 128)
    gamt = jnp.tile(gamma, 8).reshape(1, 128)
    bett = jnp.tile(beta, 8).reshape(1, 128)
    lbt = jnp.tile(lin_b, 8).reshape(1, 128)

    u_pk = _tc0(np_pad, n)(x, W1)
    degp = _deg_kernel(np_pad, ib)(dstp, ones_rows, zrows)
    t1_pk, dinv_pk = _tc1(np_pad)(u_pk, degp.reshape(_NC, nrb, 128))
    p1 = _agg_kernel(np_pad, ib)(srcp, dstp, t1_pk.reshape(np_pad, _H),
                                 zrows)
    t2_pk = _tc2(np_pad, n)(p1.reshape(_NC, nrb, 128), t1_pk, dinv_pk, b1t,
                            gamt, bett, kron8, lbt)
    q = _agg_kernel(np_pad, ib)(srcp, dstp, t2_pk.reshape(np_pad, _H), zrows)
    out = _tc3(np_pad, n, c_dim)(q.reshape(_NC, nrb, 128), t2_pk, dinv_pk,
                                 W2, b2.reshape(1, c_dim))
    return out
